# baseline (device time: 200590 ns/iter reference)
import functools

import jax
import jax.numpy as jnp
from jax import lax
from jax.experimental import pallas as pl
from jax.experimental.pallas import tpu as pltpu

N_DEV = 4
N_WTILES = 16


def kernel(x, w_mat):
    m_per, k = x.shape
    _, n_per = w_mat.shape
    half = m_per // 2

    xb = x.astype(jnp.bfloat16)

    def body(x_hbm, w_hbm, out_ref, top_ref, bot_ref, wb_ref, wtmp_ref,
             stage_ref, load_sems, wload_sem, sendR_sems, recvR_sems,
             sendL_sems, recvL_sems, copy_sems):
        my = lax.axis_index("i")
        left = (my - 1) % N_DEV
        right = (my + 1) % N_DEV
        kt = k // N_WTILES

        ld_top = pltpu.make_async_copy(
            x_hbm.at[pl.ds(0, half), :], top_ref.at[0], load_sems.at[0])
        ld_bot = pltpu.make_async_copy(
            x_hbm.at[pl.ds(half, half), :], bot_ref.at[0], load_sems.at[1])
        ld_top.start()
        ld_bot.start()

        def w_tile_copy(t):
            return pltpu.make_async_copy(
                w_hbm.at[pl.ds(t * kt, kt), :], wtmp_ref.at[t % 2],
                wload_sem.at[t % 2])

        wdmas = [w_tile_copy(0), w_tile_copy(1)]
        wdmas[0].start()
        wdmas[1].start()

        barrier_sem = pltpu.get_barrier_semaphore()
        for nbr in [left, right]:
            pl.semaphore_signal(
                barrier_sem, inc=1,
                device_id=(nbr,), device_id_type=pl.DeviceIdType.MESH,
            )
        pl.semaphore_wait(barrier_sem, 2)

        ld_top.wait()
        ld_bot.wait()

        pending = [None, None]
        counter = [0]

        nh = n_per // 2

        def do_block(buf_ref, s, row_start):
            for j in range(2):
                slot = counter[0] % 2
                counter[0] += 1
                if pending[slot] is not None:
                    pending[slot].wait()
                stage_ref[slot, :, :] = jnp.maximum(
                    jnp.dot(buf_ref[s, :, :],
                            wb_ref[:, pl.ds(j * nh, nh)],
                            preferred_element_type=jnp.float32),
                    0.0,
                )
                cp = pltpu.make_async_copy(
                    stage_ref.at[slot],
                    out_ref.at[pl.ds(row_start, half), pl.ds(j * nh, nh)],
                    copy_sems.at[slot],
                )
                cp.start()
                pending[slot] = cp

        def top_row(h):
            return ((my - h) % N_DEV) * m_per

        def bot_row(h):
            return ((my + h) % N_DEV) * m_per + half

        sends = []

        def start_hop(h):
            s, d = h % 3, (h + 1) % 3
            rR = pltpu.make_async_remote_copy(
                src_ref=top_ref.at[s], dst_ref=top_ref.at[d],
                send_sem=sendR_sems.at[h], recv_sem=recvR_sems.at[h],
                device_id=(right,), device_id_type=pl.DeviceIdType.MESH,
            )
            rL = pltpu.make_async_remote_copy(
                src_ref=bot_ref.at[s], dst_ref=bot_ref.at[d],
                send_sem=sendL_sems.at[h], recv_sem=recvL_sems.at[h],
                device_id=(left,), device_id_type=pl.DeviceIdType.MESH,
            )
            rR.start()
            rL.start()
            sends.extend([rR, rL])
            return rR, rL

        rR, rL = start_hop(0)
        for t in range(N_WTILES):
            wdmas[t].wait()
            wb_ref[pl.ds(t * kt, kt), :] = (
                wtmp_ref[t % 2, :, :].astype(jnp.bfloat16))
            if t + 2 < N_WTILES:
                wdmas.append(w_tile_copy(t + 2))
                wdmas[t + 2].start()
        do_block(top_ref, 0, top_row(0))
        rR.wait_recv()
        rL.wait_recv()

        rR, rL = start_hop(1)
        do_block(bot_ref, 0, bot_row(0))
        do_block(top_ref, 1, top_row(1))
        do_block(bot_ref, 1, bot_row(1))
        rR.wait_recv()
        rL.wait_recv()

        rR, rL = start_hop(2)
        do_block(top_ref, 2, top_row(2))
        do_block(bot_ref, 2, bot_row(2))
        rR.wait_recv()
        rL.wait_recv()

        do_block(top_ref, 0, top_row(3))
        do_block(bot_ref, 0, bot_row(3))

        for p in pending:
            if p is not None:
                p.wait()
        for snd in sends:
            snd.wait_send()

        @functools.partial(
            pl.run_scoped, second_barrier=pltpu.SemaphoreType.REGULAR)
        def _(second_barrier):
            for nbr in [left, right]:
                pl.semaphore_signal(
                    second_barrier, inc=1,
                    device_id=(nbr,), device_id_type=pl.DeviceIdType.MESH,
                )
            pl.semaphore_wait(second_barrier, 2)

    return pl.pallas_call(
        body,
        out_shape=jax.ShapeDtypeStruct((N_DEV * m_per, n_per), jnp.float32),
        in_specs=[
            pl.BlockSpec(memory_space=pl.ANY),
            pl.BlockSpec(memory_space=pl.ANY),
        ],
        out_specs=pl.BlockSpec(memory_space=pltpu.MemorySpace.HBM),
        scratch_shapes=[
            pltpu.VMEM((3, half, k), jnp.bfloat16),
            pltpu.VMEM((3, half, k), jnp.bfloat16),
            pltpu.VMEM((k, n_per), jnp.bfloat16),
            pltpu.VMEM((2, k // N_WTILES, n_per), jnp.float32),
            pltpu.VMEM((2, half, n_per // 2), jnp.float32),
            pltpu.SemaphoreType.DMA((2,)),
            pltpu.SemaphoreType.DMA((2,)),
            pltpu.SemaphoreType.DMA((N_DEV - 1,)),
            pltpu.SemaphoreType.DMA((N_DEV - 1,)),
            pltpu.SemaphoreType.DMA((N_DEV - 1,)),
            pltpu.SemaphoreType.DMA((N_DEV - 1,)),
            pltpu.SemaphoreType.DMA((2,)),
        ],
        compiler_params=pltpu.CompilerParams(
            collective_id=0,
            vmem_limit_bytes=65024 * 1024,
        ),
    )(xb, w_mat)


# device time: 187743 ns/iter; 1.0684x vs baseline; 1.0684x over previous
import functools

import jax
import jax.numpy as jnp
from jax import lax
from jax.experimental import pallas as pl
from jax.experimental.pallas import tpu as pltpu

N_DEV = 4
N_WTILES = 16


def kernel(x, w_mat):
    m_per, k = x.shape
    _, n_per = w_mat.shape
    half = m_per // 2

    xb = x.astype(jnp.bfloat16)

    def body(x_hbm, w_hbm, out_ref, top_ref, bot_ref, wb_ref, wtmp_ref,
             stage_ref, load_sems, wload_sem, sendR_sems, recvR_sems,
             sendL_sems, recvL_sems, copy_sems):
        my = lax.axis_index("i")
        left = (my - 1) % N_DEV
        right = (my + 1) % N_DEV
        kt = k // N_WTILES

        ld_top = pltpu.make_async_copy(
            x_hbm.at[pl.ds(0, half), :], top_ref.at[0], load_sems.at[0])
        ld_bot = pltpu.make_async_copy(
            x_hbm.at[pl.ds(half, half), :], bot_ref.at[0], load_sems.at[1])
        ld_top.start()
        ld_bot.start()

        def w_tile_copy(t):
            return pltpu.make_async_copy(
                w_hbm.at[pl.ds(t * kt, kt), :], wtmp_ref.at[t % 2],
                wload_sem.at[t % 2])

        wdmas = [w_tile_copy(0), w_tile_copy(1)]
        wdmas[0].start()
        wdmas[1].start()

        barrier_sem = pltpu.get_barrier_semaphore()
        for nbr in [left, right]:
            pl.semaphore_signal(
                barrier_sem, inc=1,
                device_id=(nbr,), device_id_type=pl.DeviceIdType.MESH,
            )
        pl.semaphore_wait(barrier_sem, 2)

        ld_top.wait()
        ld_bot.wait()

        pending = [None, None]
        counter = [0]

        nh = n_per // 2

        def do_rows(buf_ref, s, r0, nrows, out_row):
            for j in range(2):
                slot = counter[0] % 2
                counter[0] += 1
                if pending[slot] is not None:
                    pending[slot].wait()
                stage_ref[slot, pl.ds(0, nrows), :] = jnp.maximum(
                    jnp.dot(buf_ref[s, pl.ds(r0, nrows), :],
                            wb_ref[:, pl.ds(j * nh, nh)],
                            preferred_element_type=jnp.float32),
                    0.0,
                )
                cp = pltpu.make_async_copy(
                    stage_ref.at[slot, pl.ds(0, nrows), :],
                    out_ref.at[pl.ds(out_row, nrows), pl.ds(j * nh, nh)],
                    copy_sems.at[slot],
                )
                cp.start()
                pending[slot] = cp

        def do_block(buf_ref, s, row_start):
            do_rows(buf_ref, s, 0, half, row_start)

        def top_row(h):
            return ((my - h) % N_DEV) * m_per

        def bot_row(h):
            return ((my + h) % N_DEV) * m_per + half

        prow = half // 2
        sends = []

        def sub_send(h, p):
            s, d = h % 3, (h + 1) % 3
            r = pl.ds(p * prow, prow)
            rR = pltpu.make_async_remote_copy(
                src_ref=top_ref.at[s, r, :], dst_ref=top_ref.at[d, r, :],
                send_sem=sendR_sems.at[h, p], recv_sem=recvR_sems.at[h, p],
                device_id=(right,), device_id_type=pl.DeviceIdType.MESH,
            )
            rL = pltpu.make_async_remote_copy(
                src_ref=bot_ref.at[s, r, :], dst_ref=bot_ref.at[d, r, :],
                send_sem=sendL_sems.at[h, p], recv_sem=recvL_sems.at[h, p],
                device_id=(left,), device_id_type=pl.DeviceIdType.MESH,
            )
            rR.start()
            rL.start()
            sends.extend([rR, rL])
            return rR, rL

        h0 = [sub_send(0, 0), sub_send(0, 1)]
        for t in range(N_WTILES):
            wdmas[t].wait()
            wb_ref[pl.ds(t * kt, kt), :] = (
                wtmp_ref[t % 2, :, :].astype(jnp.bfloat16))
            if t + 2 < N_WTILES:
                wdmas.append(w_tile_copy(t + 2))
                wdmas[t + 2].start()
        do_block(top_ref, 0, top_row(0))

        for r in h0[0]:
            r.wait_recv()
        h1 = [sub_send(1, 0)]
        do_block(bot_ref, 0, bot_row(0))
        for r in h0[1]:
            r.wait_recv()
        h1.append(sub_send(1, 1))
        do_block(top_ref, 1, top_row(1))
        do_block(bot_ref, 1, bot_row(1))

        for r in h1[0]:
            r.wait_recv()
        h2 = [sub_send(2, 0)]
        for r in h1[1]:
            r.wait_recv()
        h2.append(sub_send(2, 1))
        do_block(top_ref, 2, top_row(2))
        do_block(bot_ref, 2, bot_row(2))

        for p in range(2):
            for r in h2[p]:
                r.wait_recv()
            do_rows(top_ref, 0, p * prow, prow, top_row(3) + p * prow)
            do_rows(bot_ref, 0, p * prow, prow, bot_row(3) + p * prow)

        for p in pending:
            if p is not None:
                p.wait()
        for snd in sends:
            snd.wait_send()

        @functools.partial(
            pl.run_scoped, second_barrier=pltpu.SemaphoreType.REGULAR)
        def _(second_barrier):
            for nbr in [left, right]:
                pl.semaphore_signal(
                    second_barrier, inc=1,
                    device_id=(nbr,), device_id_type=pl.DeviceIdType.MESH,
                )
            pl.semaphore_wait(second_barrier, 2)

    return pl.pallas_call(
        body,
        out_shape=jax.ShapeDtypeStruct((N_DEV * m_per, n_per), jnp.float32),
        in_specs=[
            pl.BlockSpec(memory_space=pl.ANY),
            pl.BlockSpec(memory_space=pl.ANY),
        ],
        out_specs=pl.BlockSpec(memory_space=pltpu.MemorySpace.HBM),
        scratch_shapes=[
            pltpu.VMEM((3, half, k), jnp.bfloat16),
            pltpu.VMEM((3, half, k), jnp.bfloat16),
            pltpu.VMEM((k, n_per), jnp.bfloat16),
            pltpu.VMEM((2, k // N_WTILES, n_per), jnp.float32),
            pltpu.VMEM((2, half, n_per // 2), jnp.float32),
            pltpu.SemaphoreType.DMA((2,)),
            pltpu.SemaphoreType.DMA((2,)),
            pltpu.SemaphoreType.DMA((N_DEV - 1, 2)),
            pltpu.SemaphoreType.DMA((N_DEV - 1, 2)),
            pltpu.SemaphoreType.DMA((N_DEV - 1, 2)),
            pltpu.SemaphoreType.DMA((N_DEV - 1, 2)),
            pltpu.SemaphoreType.DMA((2,)),
        ],
        compiler_params=pltpu.CompilerParams(
            collective_id=0,
            vmem_limit_bytes=65024 * 1024,
        ),
    )(xb, w_mat)


# device time: 181680 ns/iter; 1.1041x vs baseline; 1.0334x over previous
import functools

import jax
import jax.numpy as jnp
from jax import lax
from jax.experimental import pallas as pl
from jax.experimental.pallas import tpu as pltpu

N_DEV = 4
N_WTILES = 16


def kernel(x, w_mat):
    m_per, k = x.shape
    _, n_per = w_mat.shape
    half = m_per // 2

    xb = x.astype(jnp.bfloat16)

    def body(x_hbm, w_hbm, out_ref, top_ref, bot_ref, wb_ref, wtmp_ref,
             stage_ref, load_sems, wload_sem, sendR_sems, recvR_sems,
             sendL_sems, recvL_sems, copy_sems):
        my = lax.axis_index("i")
        left = (my - 1) % N_DEV
        right = (my + 1) % N_DEV
        kt = k // N_WTILES

        qr = half // 2
        lds = []
        for p in range(2):
            r = pl.ds(p * qr, qr)
            ldt = pltpu.make_async_copy(
                x_hbm.at[r, :], top_ref.at[0, r, :], load_sems.at[0, p])
            ldb = pltpu.make_async_copy(
                x_hbm.at[pl.ds(half + p * qr, qr), :], bot_ref.at[0, r, :],
                load_sems.at[1, p])
            ldt.start()
            ldb.start()
            lds.append((ldt, ldb))

        def w_tile_copy(t):
            return pltpu.make_async_copy(
                w_hbm.at[pl.ds(t * kt, kt), :], wtmp_ref.at[t % 2],
                wload_sem.at[t % 2])

        wdmas = [w_tile_copy(0), w_tile_copy(1)]
        wdmas[0].start()
        wdmas[1].start()

        barrier_sem = pltpu.get_barrier_semaphore()
        for nbr in [left, right]:
            pl.semaphore_signal(
                barrier_sem, inc=1,
                device_id=(nbr,), device_id_type=pl.DeviceIdType.MESH,
            )
        pl.semaphore_wait(barrier_sem, 2)

        pending = [None, None]
        counter = [0]

        nh = n_per // 2

        def do_rows(buf_ref, s, r0, nrows, out_row):
            for j in range(2):
                slot = counter[0] % 2
                counter[0] += 1
                if pending[slot] is not None:
                    pending[slot].wait()
                stage_ref[slot, pl.ds(0, nrows), :] = jnp.maximum(
                    jnp.dot(buf_ref[s, pl.ds(r0, nrows), :],
                            wb_ref[:, pl.ds(j * nh, nh)],
                            preferred_element_type=jnp.float32),
                    0.0,
                )
                cp = pltpu.make_async_copy(
                    stage_ref.at[slot, pl.ds(0, nrows), :],
                    out_ref.at[pl.ds(out_row, nrows), pl.ds(j * nh, nh)],
                    copy_sems.at[slot],
                )
                cp.start()
                pending[slot] = cp

        def do_block(buf_ref, s, row_start):
            do_rows(buf_ref, s, 0, half, row_start)

        def top_row(h):
            return ((my - h) % N_DEV) * m_per

        def bot_row(h):
            return ((my + h) % N_DEV) * m_per + half

        sends = []

        def sub_send(h, p, r0, nrows):
            s, d = h % 3, (h + 1) % 3
            r = pl.ds(r0, nrows)
            rR = pltpu.make_async_remote_copy(
                src_ref=top_ref.at[s, r, :], dst_ref=top_ref.at[d, r, :],
                send_sem=sendR_sems.at[h, p], recv_sem=recvR_sems.at[h, p],
                device_id=(right,), device_id_type=pl.DeviceIdType.MESH,
            )
            rL = pltpu.make_async_remote_copy(
                src_ref=bot_ref.at[s, r, :], dst_ref=bot_ref.at[d, r, :],
                send_sem=sendL_sems.at[h, p], recv_sem=recvL_sems.at[h, p],
                device_id=(left,), device_id_type=pl.DeviceIdType.MESH,
            )
            rR.start()
            rL.start()
            sends.extend([rR, rL])
            return rR, rL

        h0 = []
        for p in range(2):
            lds[p][0].wait()
            lds[p][1].wait()
            h0.append(sub_send(0, p, p * qr, qr))
        for t in range(N_WTILES):
            wdmas[t].wait()
            wb_ref[pl.ds(t * kt, kt), :] = (
                wtmp_ref[t % 2, :, :].astype(jnp.bfloat16))
            if t + 2 < N_WTILES:
                wdmas.append(w_tile_copy(t + 2))
                wdmas[t + 2].start()
        do_block(top_ref, 0, top_row(0))

        for r in h0[0]:
            r.wait_recv()
        h1 = [sub_send(1, 0, 0, qr)]
        do_block(bot_ref, 0, bot_row(0))
        for r in h0[1]:
            r.wait_recv()
        h1.append(sub_send(1, 1, qr, qr))
        do_block(top_ref, 1, top_row(1))
        do_block(bot_ref, 1, bot_row(1))

        er = qr // 2
        for r in h1[0]:
            r.wait_recv()
        h2 = [sub_send(2, 0, 0, qr)]
        for r in h1[1]:
            r.wait_recv()
        h2.append(sub_send(2, 1, qr, er))
        h2.append(sub_send(2, 2, qr + er, er))
        do_block(top_ref, 2, top_row(2))
        do_block(bot_ref, 2, bot_row(2))

        tail_pieces = [(0, qr), (qr, er), (qr + er, er)]
        for p, (r0, nrows) in enumerate(tail_pieces):
            for r in h2[p]:
                r.wait_recv()
            do_rows(top_ref, 0, r0, nrows, top_row(3) + r0)
            do_rows(bot_ref, 0, r0, nrows, bot_row(3) + r0)

        for p in pending:
            if p is not None:
                p.wait()
        for snd in sends:
            snd.wait_send()

        @functools.partial(
            pl.run_scoped, second_barrier=pltpu.SemaphoreType.REGULAR)
        def _(second_barrier):
            for nbr in [left, right]:
                pl.semaphore_signal(
                    second_barrier, inc=1,
                    device_id=(nbr,), device_id_type=pl.DeviceIdType.MESH,
                )
            pl.semaphore_wait(second_barrier, 2)

    return pl.pallas_call(
        body,
        out_shape=jax.ShapeDtypeStruct((N_DEV * m_per, n_per), jnp.float32),
        in_specs=[
            pl.BlockSpec(memory_space=pl.ANY),
            pl.BlockSpec(memory_space=pl.ANY),
        ],
        out_specs=pl.BlockSpec(memory_space=pltpu.MemorySpace.HBM),
        scratch_shapes=[
            pltpu.VMEM((3, half, k), jnp.bfloat16),
            pltpu.VMEM((3, half, k), jnp.bfloat16),
            pltpu.VMEM((k, n_per), jnp.bfloat16),
            pltpu.VMEM((2, k // N_WTILES, n_per), jnp.float32),
            pltpu.VMEM((2, half, n_per // 2), jnp.float32),
            pltpu.SemaphoreType.DMA((2, 2)),
            pltpu.SemaphoreType.DMA((2,)),
            pltpu.SemaphoreType.DMA((N_DEV - 1, 3)),
            pltpu.SemaphoreType.DMA((N_DEV - 1, 3)),
            pltpu.SemaphoreType.DMA((N_DEV - 1, 3)),
            pltpu.SemaphoreType.DMA((N_DEV - 1, 3)),
            pltpu.SemaphoreType.DMA((2,)),
        ],
        compiler_params=pltpu.CompilerParams(
            collective_id=0,
            vmem_limit_bytes=65024 * 1024,
        ),
    )(xb, w_mat)
